# baseline (speedup 1.0000x reference)
import functools

import jax
import jax.numpy as jnp
from jax import lax
from jax.experimental import pallas as pl
from jax.experimental.pallas import tpu as pltpu
from jax.experimental.pallas import tpu_sc as plsc

_VOCAB = 1000000
_D = 64
_HIST = 50
_BATCH = 16384
_HID = 256
_NUM_CLASSES = 2

_NC = 2
_NS = 16
_NW = _NC * _NS
_ROWS_PER_W = _BATCH // _NW
_CHUNK_ROWS = 2
_CHUNK_IDX = _CHUNK_ROWS * _HIST
_NCHUNK = _ROWS_PER_W // _CHUNK_ROWS
_NBUF = 2

_CCHUNK = 512
_NCC = _VOCAB // _CCHUNK
_CTAIL = _VOCAB - _NCC * _CCHUNK
_CC_PER_W = (_NCC + _NW - 1) // _NW

_COL_PERM = tuple(
    32 * s + off for s in range(_D // 32)
    for off in list(range(0, 32, 2)) + list(range(1, 32, 2))
)


def _mesh():
  return plsc.VectorSubcoreMesh(
      core_axis_name="c", subcore_axis_name="s",
      num_cores=_NC, num_subcores=_NS)


def _sc_convert(tableT, tail_pad):
  @functools.partial(
      pl.kernel,
      out_type=jax.ShapeDtypeStruct((_VOCAB // 4, 128), jnp.float32),
      mesh=_mesh(),
      scratch_types=[
          pltpu.VMEM((_NBUF, _D, _CCHUNK), jnp.float32),
          pltpu.VMEM((_CCHUNK // 4, 128), jnp.float32),
          pltpu.SemaphoreType.DMA((_NBUF,)),
      ],
      compiler_params=pltpu.CompilerParams(needs_layout_passes=False),
  )
  def a_kernel(t_hbm, tail_hbm, out_hbm, in_v, out_v, isems):
    wid = lax.axis_index("c") * _NS + lax.axis_index("s")
    row_ids = lax.broadcasted_iota(jnp.int32, (16,), 0)
    flat_base = row_ids * (_D // 2)

    def issue(k, b):
      c = wid + k * _NW
      @pl.when(c < _NCC)
      def _():
        pltpu.async_copy(t_hbm.at[:, pl.ds(c * _CCHUNK, _CCHUNK)],
                         in_v.at[b], isems.at[b])

    def transpose_chunk(b, nrows):
      def rne_hi(u):
        return u + jnp.uint32(0x7FFF) + lax.bitwise_and(
            lax.shift_right_logical(u, jnp.uint32(16)), jnp.uint32(1))

      def step(g, _):
        k0 = g * 16
        for cp in range(_D // 2):
          a = in_v[b, 2 * cp, pl.ds(k0, 16)]
          bb = in_v[b, 2 * cp + 1, pl.ds(k0, 16)]
          ra = rne_hi(plsc.bitcast(a, jnp.uint32))
          rb = rne_hi(plsc.bitcast(bb, jnp.uint32))
          word = lax.bitwise_or(
              lax.shift_right_logical(ra, jnp.uint32(16)),
              lax.bitwise_and(rb, jnp.uint32(0xFFFF0000)))
          w = plsc.bitcast(word, jnp.float32)
          flat = flat_base + (k0 * (_D // 2) + cp)
          plsc.store_scatter(
              out_v, [lax.shift_right_logical(flat, 7),
                      lax.bitwise_and(flat, 127)], w)
        return 0
      lax.fori_loop(0, nrows // 16, step, 0, unroll=False)

    for b in range(_NBUF):
      issue(b, b)

    def body(kk, b):
      c = wid + kk * _NW
      @pl.when(c < _NCC)
      def _():
        pltpu.make_async_copy(t_hbm.at[:, pl.ds(0, _CCHUNK)], in_v.at[b],
                              isems.at[b]).wait()
        transpose_chunk(b, _CCHUNK)
        issue(kk + _NBUF, b)
        pltpu.sync_copy(out_v, out_hbm.at[pl.ds(c * (_CCHUNK // 4),
                                                _CCHUNK // 4)])

    lax.fori_loop(
        0, _CC_PER_W // _NBUF,
        lambda i, _: ([body(i * _NBUF + b, b) for b in range(_NBUF)], 0)[1],
        0, unroll=False)
    for kk in range((_CC_PER_W // _NBUF) * _NBUF, _CC_PER_W):
      body(kk, kk % _NBUF)
    @pl.when(wid == 1)
    def _():
      pltpu.sync_copy(tail_hbm, in_v.at[0, :, pl.ds(0, 128)])
      transpose_chunk(0, _CTAIL)
      pltpu.sync_copy(out_v.at[pl.ds(0, _CTAIL // 4)],
                      out_hbm.at[pl.ds(_NCC * (_CCHUNK // 4), _CTAIL // 4)])

  return a_kernel(tableT, tail_pad)


def _sc_gather_mean(table_pairs, x_chunks):
  @functools.partial(
      pl.kernel,
      out_type=jax.ShapeDtypeStruct((_BATCH, _D), jnp.float32),
      mesh=_mesh(),
      scratch_types=[
          pltpu.VMEM((_NCHUNK, _CHUNK_IDX), jnp.int32),
          pltpu.VMEM((_NBUF, _CHUNK_IDX, _D // 2), jnp.float32),
          pltpu.VMEM((_ROWS_PER_W, _D), jnp.float32),
          pltpu.SemaphoreType.DMA((_NBUF,)),
      ],
      compiler_params=pltpu.CompilerParams(
          use_tc_tiling_on_sc=False, needs_layout_passes=False),
  )
  def sc_kernel(table_hbm, x_hbm, out_hbm, idx_v, buf_v, out_v, sems):
    wid = lax.axis_index("c") * _NS + lax.axis_index("s")
    chunk_base = wid * _NCHUNK
    pltpu.sync_copy(x_hbm.at[pl.ds(chunk_base, _NCHUNK)], idx_v)

    def issue(j, b):
      pltpu.async_copy(table_hbm.at[idx_v.at[j]], buf_v.at[b], sems.at[b])

    for b in range(_NBUF):
      issue(b, b)

    def drain_wait(b):
      pltpu.make_async_copy(table_hbm.at[idx_v.at[0]], buf_v.at[b],
                            sems.at[b]).wait()

    ngrp = _D // 32

    def body(j):
      for b in range(_NBUF):
        jj = j + b
        drain_wait(b)
        for r2 in range(_CHUNK_ROWS):
          acc_e = [None] * ngrp
          acc_o = [None] * ngrp
          for r in range(_HIST):
            for s in range(ngrp):
              w = buf_v[b, r2 * _HIST + r, pl.ds(s * 16, 16)]
              v = plsc.bitcast(w, jnp.bfloat16)
              e, o = plsc.unpack(v, format=plsc.PackFormat.INTERLEAVED)
              if r == 0:
                acc_e[s], acc_o[s] = e, o
              else:
                acc_e[s] = acc_e[s] + e
                acc_o[s] = acc_o[s] + o
          row = jj * _CHUNK_ROWS + r2
          for s in range(ngrp):
            out_v[row, pl.ds(32 * s, 16)] = acc_e[s] * (1.0 / _HIST)
            out_v[row, pl.ds(32 * s + 16, 16)] = acc_o[s] * (1.0 / _HIST)
        nxt = jj + _NBUF
        @pl.when(nxt < _NCHUNK)
        def _():
          issue(nxt, b)

    lax.fori_loop(0, _NCHUNK // _NBUF, lambda i, _: (body(i * _NBUF), 0)[1], 0,
                  unroll=False)
    pltpu.sync_copy(out_v, out_hbm.at[pl.ds(wid * _ROWS_PER_W, _ROWS_PER_W)])

  return sc_kernel(table_pairs, x_chunks)


def _tc_mlp(averaged, V_w, V_b, W_wp, W_bp):
  blk = 1024

  def mlp_kernel(avg_ref, vw_ref, vb_ref, ww_ref, wb_ref, out_ref):
    h = jnp.dot(avg_ref[...], vw_ref[...], preferred_element_type=jnp.float32)
    h = jnp.maximum(h + vb_ref[...], 0.0)
    logits = jnp.dot(h, ww_ref[...], preferred_element_type=jnp.float32)
    logits = logits + wb_ref[...]
    col = lax.broadcasted_iota(jnp.int32, logits.shape, 1)
    valid = col < _NUM_CLASSES
    neg = jnp.full_like(logits, -jnp.inf)
    masked = jnp.where(valid, logits, neg)
    m = jnp.max(masked, axis=1, keepdims=True)
    ex = jnp.where(valid, jnp.exp(masked - m), 0.0)
    lse = jnp.log(jnp.sum(ex, axis=1, keepdims=True)) + m
    out_ref[...] = logits - lse

  grid = _BATCH // blk
  return pl.pallas_call(
      mlp_kernel,
      grid=(grid,),
      in_specs=[
          pl.BlockSpec((blk, _D), lambda i: (i, 0)),
          pl.BlockSpec((_D, _HID), lambda i: (0, 0)),
          pl.BlockSpec((1, _HID), lambda i: (0, 0)),
          pl.BlockSpec((_HID, 128), lambda i: (0, 0)),
          pl.BlockSpec((1, 128), lambda i: (0, 0)),
      ],
      out_specs=pl.BlockSpec((blk, 128), lambda i: (i, 0)),
      out_shape=jax.ShapeDtypeStruct((_BATCH, 128), jnp.float32),
  )(averaged, V_w, V_b, W_wp, W_bp)


def kernel(x, table, V_w, V_b, W_w, W_b):
  x_chunks = x.reshape(_NW * _NCHUNK, _CHUNK_IDX)
  tail_pad = jnp.pad(table[_NCC * _CCHUNK:, :].T,
                     ((0, 0), (0, 128 - _CTAIL)))
  packed = _sc_convert(table.T, tail_pad)
  table_pairs = packed.reshape(_VOCAB, _D // 2)
  averaged_p = _sc_gather_mean(table_pairs, x_chunks)
  V_w_p = V_w[jnp.array(_COL_PERM, dtype=jnp.int32), :]
  W_wp = jnp.pad(W_w, ((0, 0), (0, 128 - _NUM_CLASSES)))
  W_bp = jnp.pad(W_b, (0, 128 - _NUM_CLASSES)).reshape(1, 128)
  out_full = _tc_mlp(averaged_p, V_w_p, V_b.reshape(1, _HID), W_wp, W_bp)
  return out_full[:, :_NUM_CLASSES]


# speedup vs baseline: 1.0005x; 1.0005x over previous
import functools

import jax
import jax.numpy as jnp
from jax import lax
from jax.experimental import pallas as pl
from jax.experimental.pallas import tpu as pltpu
from jax.experimental.pallas import tpu_sc as plsc

_VOCAB = 1000000
_D = 64
_HIST = 50
_BATCH = 16384
_HID = 256
_NUM_CLASSES = 2

_NC = 2
_NS = 16
_NW = _NC * _NS
_ROWS_PER_W = _BATCH // _NW
_CHUNK_ROWS = 2
_CHUNK_IDX = _CHUNK_ROWS * _HIST
_NCHUNK = _ROWS_PER_W // _CHUNK_ROWS
_NBUF = 2

_CCHUNK = 512
_NCC = _VOCAB // _CCHUNK
_CTAIL = _VOCAB - _NCC * _CCHUNK
_CC_PER_W = (_NCC + _NW - 1) // _NW

_COL_PERM = tuple(
    32 * s + off for s in range(_D // 32)
    for off in list(range(0, 32, 2)) + list(range(1, 32, 2))
)


def _mesh():
  return plsc.VectorSubcoreMesh(
      core_axis_name="c", subcore_axis_name="s",
      num_cores=_NC, num_subcores=_NS)


def _sc_convert(tableT, tail_pad):
  @functools.partial(
      pl.kernel,
      out_type=jax.ShapeDtypeStruct((_VOCAB // 4, 128), jnp.float32),
      mesh=_mesh(),
      scratch_types=[
          pltpu.VMEM((_NBUF, _D, _CCHUNK), jnp.float32),
          pltpu.VMEM((_CCHUNK // 4, 132), jnp.float32),
          pltpu.SemaphoreType.DMA((_NBUF,)),
      ],
      compiler_params=pltpu.CompilerParams(needs_layout_passes=False),
  )
  def a_kernel(t_hbm, tail_hbm, out_hbm, in_v, out_v, isems):
    wid = lax.axis_index("c") * _NS + lax.axis_index("s")
    row_ids = lax.broadcasted_iota(jnp.int32, (16,), 0)
    flat_base = row_ids * (_D // 2)

    def issue(k, b):
      c = wid + k * _NW
      @pl.when(c < _NCC)
      def _():
        pltpu.async_copy(t_hbm.at[:, pl.ds(c * _CCHUNK, _CCHUNK)],
                         in_v.at[b], isems.at[b])

    def transpose_chunk(b, nrows):
      def rne_hi(u):
        return u + jnp.uint32(0x7FFF) + lax.bitwise_and(
            lax.shift_right_logical(u, jnp.uint32(16)), jnp.uint32(1))

      def step(g, _):
        k0 = g * 16
        for cp in range(_D // 2):
          a = in_v[b, 2 * cp, pl.ds(k0, 16)]
          bb = in_v[b, 2 * cp + 1, pl.ds(k0, 16)]
          ra = rne_hi(plsc.bitcast(a, jnp.uint32))
          rb = rne_hi(plsc.bitcast(bb, jnp.uint32))
          word = lax.bitwise_or(
              lax.shift_right_logical(ra, jnp.uint32(16)),
              lax.bitwise_and(rb, jnp.uint32(0xFFFF0000)))
          w = plsc.bitcast(word, jnp.float32)
          flat = flat_base + (k0 * (_D // 2) + cp)
          plsc.store_scatter(
              out_v, [lax.shift_right_logical(flat, 7),
                      lax.bitwise_and(flat, 127)], w)
        return 0
      lax.fori_loop(0, nrows // 16, step, 0, unroll=False)

    for b in range(_NBUF):
      issue(b, b)

    def body(kk, b):
      c = wid + kk * _NW
      @pl.when(c < _NCC)
      def _():
        pltpu.make_async_copy(t_hbm.at[:, pl.ds(0, _CCHUNK)], in_v.at[b],
                              isems.at[b]).wait()
        transpose_chunk(b, _CCHUNK)
        issue(kk + _NBUF, b)
        pltpu.sync_copy(out_v.at[:, pl.ds(0, 128)],
                        out_hbm.at[pl.ds(c * (_CCHUNK // 4), _CCHUNK // 4)])

    lax.fori_loop(
        0, _CC_PER_W // _NBUF,
        lambda i, _: ([body(i * _NBUF + b, b) for b in range(_NBUF)], 0)[1],
        0, unroll=False)
    for kk in range((_CC_PER_W // _NBUF) * _NBUF, _CC_PER_W):
      body(kk, kk % _NBUF)
    @pl.when(wid == 1)
    def _():
      pltpu.sync_copy(tail_hbm, in_v.at[0, :, pl.ds(0, 128)])
      transpose_chunk(0, _CTAIL)
      pltpu.sync_copy(out_v.at[pl.ds(0, _CTAIL // 4), pl.ds(0, 128)],
                      out_hbm.at[pl.ds(_NCC * (_CCHUNK // 4), _CTAIL // 4)])

  return a_kernel(tableT, tail_pad)


def _sc_gather_mean(table_pairs, x_chunks):
  @functools.partial(
      pl.kernel,
      out_type=jax.ShapeDtypeStruct((_BATCH, _D), jnp.float32),
      mesh=_mesh(),
      scratch_types=[
          pltpu.VMEM((_NCHUNK, _CHUNK_IDX), jnp.int32),
          pltpu.VMEM((_NBUF, _CHUNK_IDX, _D // 2), jnp.float32),
          pltpu.VMEM((_ROWS_PER_W, _D), jnp.float32),
          pltpu.SemaphoreType.DMA((_NBUF,)),
      ],
      compiler_params=pltpu.CompilerParams(
          use_tc_tiling_on_sc=False, needs_layout_passes=False),
  )
  def sc_kernel(table_hbm, x_hbm, out_hbm, idx_v, buf_v, out_v, sems):
    wid = lax.axis_index("c") * _NS + lax.axis_index("s")
    chunk_base = wid * _NCHUNK
    pltpu.sync_copy(x_hbm.at[pl.ds(chunk_base, _NCHUNK)], idx_v)

    def issue(j, b):
      pltpu.async_copy(table_hbm.at[idx_v.at[j]], buf_v.at[b], sems.at[b])

    for b in range(_NBUF):
      issue(b, b)

    def drain_wait(b):
      pltpu.make_async_copy(table_hbm.at[idx_v.at[0]], buf_v.at[b],
                            sems.at[b]).wait()

    ngrp = _D // 32

    def body(j):
      for b in range(_NBUF):
        jj = j + b
        drain_wait(b)
        for r2 in range(_CHUNK_ROWS):
          acc_e = [None] * ngrp
          acc_o = [None] * ngrp
          for r in range(_HIST):
            for s in range(ngrp):
              w = buf_v[b, r2 * _HIST + r, pl.ds(s * 16, 16)]
              v = plsc.bitcast(w, jnp.bfloat16)
              e, o = plsc.unpack(v, format=plsc.PackFormat.INTERLEAVED)
              if r == 0:
                acc_e[s], acc_o[s] = e, o
              else:
                acc_e[s] = acc_e[s] + e
                acc_o[s] = acc_o[s] + o
          row = jj * _CHUNK_ROWS + r2
          for s in range(ngrp):
            out_v[row, pl.ds(32 * s, 16)] = acc_e[s] * (1.0 / _HIST)
            out_v[row, pl.ds(32 * s + 16, 16)] = acc_o[s] * (1.0 / _HIST)
        nxt = jj + _NBUF
        @pl.when(nxt < _NCHUNK)
        def _():
          issue(nxt, b)

    lax.fori_loop(0, _NCHUNK // _NBUF, lambda i, _: (body(i * _NBUF), 0)[1], 0,
                  unroll=False)
    pltpu.sync_copy(out_v, out_hbm.at[pl.ds(wid * _ROWS_PER_W, _ROWS_PER_W)])

  return sc_kernel(table_pairs, x_chunks)


def _tc_mlp(averaged, V_w, V_b, W_wp, W_bp):
  blk = 1024

  def mlp_kernel(avg_ref, vw_ref, vb_ref, ww_ref, wb_ref, out_ref):
    h = jnp.dot(avg_ref[...], vw_ref[...], preferred_element_type=jnp.float32)
    h = jnp.maximum(h + vb_ref[...], 0.0)
    logits = jnp.dot(h, ww_ref[...], preferred_element_type=jnp.float32)
    logits = logits + wb_ref[...]
    col = lax.broadcasted_iota(jnp.int32, logits.shape, 1)
    valid = col < _NUM_CLASSES
    neg = jnp.full_like(logits, -jnp.inf)
    masked = jnp.where(valid, logits, neg)
    m = jnp.max(masked, axis=1, keepdims=True)
    ex = jnp.where(valid, jnp.exp(masked - m), 0.0)
    lse = jnp.log(jnp.sum(ex, axis=1, keepdims=True)) + m
    out_ref[...] = logits - lse

  grid = _BATCH // blk
  return pl.pallas_call(
      mlp_kernel,
      grid=(grid,),
      in_specs=[
          pl.BlockSpec((blk, _D), lambda i: (i, 0)),
          pl.BlockSpec((_D, _HID), lambda i: (0, 0)),
          pl.BlockSpec((1, _HID), lambda i: (0, 0)),
          pl.BlockSpec((_HID, 128), lambda i: (0, 0)),
          pl.BlockSpec((1, 128), lambda i: (0, 0)),
      ],
      out_specs=pl.BlockSpec((blk, 128), lambda i: (i, 0)),
      out_shape=jax.ShapeDtypeStruct((_BATCH, 128), jnp.float32),
  )(averaged, V_w, V_b, W_wp, W_bp)


def kernel(x, table, V_w, V_b, W_w, W_b):
  x_chunks = x.reshape(_NW * _NCHUNK, _CHUNK_IDX)
  tail_pad = jnp.pad(table[_NCC * _CCHUNK:, :].T,
                     ((0, 0), (0, 128 - _CTAIL)))
  packed = _sc_convert(table.T, tail_pad)
  table_pairs = packed.reshape(_VOCAB, _D // 2)
  averaged_p = _sc_gather_mean(table_pairs, x_chunks)
  V_w_p = V_w[jnp.array(_COL_PERM, dtype=jnp.int32), :]
  W_wp = jnp.pad(W_w, ((0, 0), (0, 128 - _NUM_CLASSES)))
  W_bp = jnp.pad(W_b, (0, 128 - _NUM_CLASSES)).reshape(1, 128)
  out_full = _tc_mlp(averaged_p, V_w_p, V_b.reshape(1, _HID), W_wp, W_bp)
  return out_full[:, :_NUM_CLASSES]


# speedup vs baseline: 1.1983x; 1.1976x over previous
import functools

import jax
import jax.numpy as jnp
from jax import lax
from jax.experimental import pallas as pl
from jax.experimental.pallas import tpu as pltpu
from jax.experimental.pallas import tpu_sc as plsc

_VOCAB = 1000000
_D = 64
_HIST = 50
_BATCH = 16384
_HID = 256
_NUM_CLASSES = 2

_NC = 2
_NS = 16
_NW = _NC * _NS
_ROWS_PER_W = _BATCH // _NW
_CHUNK_ROWS = 2
_CHUNK_IDX = _CHUNK_ROWS * _HIST
_NCHUNK = _ROWS_PER_W // _CHUNK_ROWS
_NBUF = 2

_CCHUNK = 512
_NCC = _VOCAB // _CCHUNK
_CTAIL = _VOCAB - _NCC * _CCHUNK
_CC_PER_W = (_NCC + _NW - 1) // _NW

_COL_PERM = tuple(
    32 * s + off for s in range(_D // 32)
    for off in list(range(0, 32, 2)) + list(range(1, 32, 2))
)


def _mesh():
  return plsc.VectorSubcoreMesh(
      core_axis_name="c", subcore_axis_name="s",
      num_cores=_NC, num_subcores=_NS)


def _sc_convert(tableT, tail_pad):
  @functools.partial(
      pl.kernel,
      out_type=jax.ShapeDtypeStruct((_VOCAB // 4, 128), jnp.float32),
      mesh=_mesh(),
      scratch_types=[
          pltpu.VMEM((_NBUF, _D, _CCHUNK), jnp.float32),
          pltpu.VMEM((_CCHUNK // 4, 132), jnp.float32),
          pltpu.SemaphoreType.DMA((_NBUF,)),
      ],
      compiler_params=pltpu.CompilerParams(needs_layout_passes=False),
  )
  def a_kernel(t_hbm, tail_hbm, out_hbm, in_v, out_v, isems):
    wid = lax.axis_index("c") * _NS + lax.axis_index("s")
    row_ids = lax.broadcasted_iota(jnp.int32, (16,), 0)
    flat_base = row_ids * (_D // 2)

    def issue(k, b):
      c = wid + k * _NW
      @pl.when(c < _NCC)
      def _():
        pltpu.async_copy(t_hbm.at[:, pl.ds(c * _CCHUNK, _CCHUNK)],
                         in_v.at[b], isems.at[b])

    def transpose_chunk(b, nrows):
      def step(g, _):
        k0 = g * 16
        for cp in range(_D // 2):
          a = in_v[b, 2 * cp, pl.ds(k0, 16)]
          bb = in_v[b, 2 * cp + 1, pl.ds(k0, 16)]
          p = plsc.pack(a, bb, format=plsc.PackFormat.INTERLEAVED)
          w = plsc.bitcast(p, jnp.float32)
          flat = flat_base + (k0 * (_D // 2) + cp)
          plsc.store_scatter(
              out_v, [lax.shift_right_logical(flat, 7),
                      lax.bitwise_and(flat, 127)], w)
        return 0
      lax.fori_loop(0, nrows // 16, step, 0, unroll=False)

    for b in range(_NBUF):
      issue(b, b)

    def body(kk, b):
      c = wid + kk * _NW
      @pl.when(c < _NCC)
      def _():
        pltpu.make_async_copy(t_hbm.at[:, pl.ds(0, _CCHUNK)], in_v.at[b],
                              isems.at[b]).wait()
        transpose_chunk(b, _CCHUNK)
        issue(kk + _NBUF, b)
        pltpu.sync_copy(out_v.at[:, pl.ds(0, 128)],
                        out_hbm.at[pl.ds(c * (_CCHUNK // 4), _CCHUNK // 4)])

    lax.fori_loop(
        0, _CC_PER_W // _NBUF,
        lambda i, _: ([body(i * _NBUF + b, b) for b in range(_NBUF)], 0)[1],
        0, unroll=False)
    for kk in range((_CC_PER_W // _NBUF) * _NBUF, _CC_PER_W):
      body(kk, kk % _NBUF)
    @pl.when(wid == 1)
    def _():
      pltpu.sync_copy(tail_hbm, in_v.at[0, :, pl.ds(0, 128)])
      transpose_chunk(0, _CTAIL)
      pltpu.sync_copy(out_v.at[pl.ds(0, _CTAIL // 4), pl.ds(0, 128)],
                      out_hbm.at[pl.ds(_NCC * (_CCHUNK // 4), _CTAIL // 4)])

  return a_kernel(tableT, tail_pad)


def _sc_gather_mean(table_pairs, x_chunks):
  @functools.partial(
      pl.kernel,
      out_type=jax.ShapeDtypeStruct((_BATCH, _D), jnp.float32),
      mesh=_mesh(),
      scratch_types=[
          pltpu.VMEM((_NCHUNK, _CHUNK_IDX), jnp.int32),
          pltpu.VMEM((_NBUF, _CHUNK_IDX, _D // 2), jnp.float32),
          pltpu.VMEM((_ROWS_PER_W, _D), jnp.float32),
          pltpu.SemaphoreType.DMA((_NBUF,)),
      ],
      compiler_params=pltpu.CompilerParams(
          use_tc_tiling_on_sc=False, needs_layout_passes=False),
  )
  def sc_kernel(table_hbm, x_hbm, out_hbm, idx_v, buf_v, out_v, sems):
    wid = lax.axis_index("c") * _NS + lax.axis_index("s")
    chunk_base = wid * _NCHUNK
    pltpu.sync_copy(x_hbm.at[pl.ds(chunk_base, _NCHUNK)], idx_v)

    def issue(j, b):
      pltpu.async_copy(table_hbm.at[idx_v.at[j]], buf_v.at[b], sems.at[b])

    for b in range(_NBUF):
      issue(b, b)

    def drain_wait(b):
      pltpu.make_async_copy(table_hbm.at[idx_v.at[0]], buf_v.at[b],
                            sems.at[b]).wait()

    ngrp = _D // 32

    def body(j):
      for b in range(_NBUF):
        jj = j + b
        drain_wait(b)
        for r2 in range(_CHUNK_ROWS):
          acc_e = [None] * ngrp
          acc_o = [None] * ngrp
          for r in range(_HIST):
            for s in range(ngrp):
              w = buf_v[b, r2 * _HIST + r, pl.ds(s * 16, 16)]
              v = plsc.bitcast(w, jnp.bfloat16)
              e, o = plsc.unpack(v, format=plsc.PackFormat.INTERLEAVED)
              if r == 0:
                acc_e[s], acc_o[s] = e, o
              else:
                acc_e[s] = acc_e[s] + e
                acc_o[s] = acc_o[s] + o
          row = jj * _CHUNK_ROWS + r2
          for s in range(ngrp):
            out_v[row, pl.ds(32 * s, 16)] = acc_e[s] * (1.0 / _HIST)
            out_v[row, pl.ds(32 * s + 16, 16)] = acc_o[s] * (1.0 / _HIST)
        nxt = jj + _NBUF
        @pl.when(nxt < _NCHUNK)
        def _():
          issue(nxt, b)

    lax.fori_loop(0, _NCHUNK // _NBUF, lambda i, _: (body(i * _NBUF), 0)[1], 0,
                  unroll=False)
    pltpu.sync_copy(out_v, out_hbm.at[pl.ds(wid * _ROWS_PER_W, _ROWS_PER_W)])

  return sc_kernel(table_pairs, x_chunks)


def _tc_mlp(averaged, V_w, V_b, W_wp, W_bp):
  blk = 1024

  def mlp_kernel(avg_ref, vw_ref, vb_ref, ww_ref, wb_ref, out_ref):
    h = jnp.dot(avg_ref[...], vw_ref[...], preferred_element_type=jnp.float32)
    h = jnp.maximum(h + vb_ref[...], 0.0)
    logits = jnp.dot(h, ww_ref[...], preferred_element_type=jnp.float32)
    logits = logits + wb_ref[...]
    col = lax.broadcasted_iota(jnp.int32, logits.shape, 1)
    valid = col < _NUM_CLASSES
    neg = jnp.full_like(logits, -jnp.inf)
    masked = jnp.where(valid, logits, neg)
    m = jnp.max(masked, axis=1, keepdims=True)
    ex = jnp.where(valid, jnp.exp(masked - m), 0.0)
    lse = jnp.log(jnp.sum(ex, axis=1, keepdims=True)) + m
    out_ref[...] = logits - lse

  grid = _BATCH // blk
  return pl.pallas_call(
      mlp_kernel,
      grid=(grid,),
      in_specs=[
          pl.BlockSpec((blk, _D), lambda i: (i, 0)),
          pl.BlockSpec((_D, _HID), lambda i: (0, 0)),
          pl.BlockSpec((1, _HID), lambda i: (0, 0)),
          pl.BlockSpec((_HID, 128), lambda i: (0, 0)),
          pl.BlockSpec((1, 128), lambda i: (0, 0)),
      ],
      out_specs=pl.BlockSpec((blk, 128), lambda i: (i, 0)),
      out_shape=jax.ShapeDtypeStruct((_BATCH, 128), jnp.float32),
  )(averaged, V_w, V_b, W_wp, W_bp)


def kernel(x, table, V_w, V_b, W_w, W_b):
  x_chunks = x.reshape(_NW * _NCHUNK, _CHUNK_IDX)
  tail_pad = jnp.pad(table[_NCC * _CCHUNK:, :].T,
                     ((0, 0), (0, 128 - _CTAIL)))
  packed = _sc_convert(table.T, tail_pad)
  table_pairs = packed.reshape(_VOCAB, _D // 2)
  averaged_p = _sc_gather_mean(table_pairs, x_chunks)
  V_w_p = V_w[jnp.array(_COL_PERM, dtype=jnp.int32), :]
  W_wp = jnp.pad(W_w, ((0, 0), (0, 128 - _NUM_CLASSES)))
  W_bp = jnp.pad(W_b, (0, 128 - _NUM_CLASSES)).reshape(1, 128)
  out_full = _tc_mlp(averaged_p, V_w_p, V_b.reshape(1, _HID), W_wp, W_bp)
  return out_full[:, :_NUM_CLASSES]
